# baseline (speedup 1.0000x reference)
import jax
import jax.numpy as jnp
from jax.experimental import pallas as pl


_ROWS_BLK = 8


def _rowsum_body(x_ref, o_ref):
    o_ref[...] = jnp.sum(x_ref[...], axis=1, keepdims=True)


def _gate_body(kkeep, m_ref, w1_ref, w2_ref, o_ref):
    y = m_ref[...]
    b, c = y.shape
    h = jax.lax.dot_general(y, w1_ref[...], (((1,), (1,)), ((), ())),
                            preferred_element_type=jnp.float32)
    h = jnp.maximum(h, 0.0)
    z = jax.lax.dot_general(h, w2_ref[...], (((1,), (1,)), ((), ())),
                            preferred_element_type=jnp.float32)
    g = jax.nn.sigmoid(z)

    gbits = jax.lax.bitcast_convert_type(g, jnp.int32)
    lo = jnp.zeros((b, 1), jnp.int32)
    hi = jnp.full((b, 1), 0x3F800000, jnp.int32)

    def body(_, lohi):
        lo, hi = lohi
        mid = lo + (hi - lo + 1) // 2
        cnt = jnp.sum((gbits >= mid).astype(jnp.int32), axis=1, keepdims=True)
        ok = cnt >= kkeep
        return jnp.where(ok, mid, lo), jnp.where(ok, hi, mid - 1)

    lo, hi = jax.lax.fori_loop(0, 31, body, (lo, hi))
    vbits = lo

    gt = (gbits > vbits).astype(jnp.float32)
    eq = (gbits == vbits).astype(jnp.float32)
    n_gt = jnp.sum(gt, axis=1, keepdims=True)
    need = jnp.float32(kkeep) - n_gt

    ii = jax.lax.broadcasted_iota(jnp.int32, (c, c), 0)
    jj = jax.lax.broadcasted_iota(jnp.int32, (c, c), 1)
    ltri = (ii < jj).astype(jnp.float32)
    prefix = jax.lax.dot_general(eq, ltri, (((1,), (0,)), ((), ())),
                                 preferred_element_type=jnp.float32)
    keep = gt + eq * (prefix < need).astype(jnp.float32)
    o_ref[...] = g * keep


def _scale_body(x_ref, g_ref, o_ref):
    o_ref[...] = x_ref[...] * g_ref[...]


def kernel(x, W1, W2):
    b, c, h, w = x.shape
    hw = h * w
    bc = b * c
    kkeep = c // 2
    xr = x.reshape(bc, hw)

    sums = pl.pallas_call(
        _rowsum_body,
        grid=(bc // _ROWS_BLK,),
        in_specs=[pl.BlockSpec((_ROWS_BLK, hw), lambda i: (i, 0))],
        out_specs=pl.BlockSpec((_ROWS_BLK, 1), lambda i: (i, 0)),
        out_shape=jax.ShapeDtypeStruct((bc, 1), jnp.float32),
    )(xr)
    means = sums.reshape(b, c) * (1.0 / hw)

    gate = pl.pallas_call(
        lambda *refs: _gate_body(kkeep, *refs),
        in_specs=[pl.BlockSpec((b, c), lambda: (0, 0)),
                  pl.BlockSpec(W1.shape, lambda: (0, 0)),
                  pl.BlockSpec(W2.shape, lambda: (0, 0))],
        out_specs=pl.BlockSpec((b, c), lambda: (0, 0)),
        out_shape=jax.ShapeDtypeStruct((b, c), jnp.float32),
    )(means, W1, W2)

    out = pl.pallas_call(
        _scale_body,
        grid=(bc // _ROWS_BLK,),
        in_specs=[pl.BlockSpec((_ROWS_BLK, hw), lambda i: (i, 0)),
                  pl.BlockSpec((_ROWS_BLK, 1), lambda i: (i, 0))],
        out_specs=pl.BlockSpec((_ROWS_BLK, hw), lambda i: (i, 0)),
        out_shape=jax.ShapeDtypeStruct((bc, hw), jnp.float32),
    )(xr, gate.reshape(bc, 1))

    return out.reshape(b, c, h, w)


# speedup vs baseline: 1.8347x; 1.8347x over previous
import jax
import jax.numpy as jnp
from jax.experimental import pallas as pl


_ROWS_BLK = 16


def _rowsum_body(x_ref, o_ref):
    o_ref[...] = jnp.sum(x_ref[...], axis=(1, 2), keepdims=True)


def _gate_body(kkeep, m_ref, w1_ref, w2_ref, o_ref):
    y = m_ref[...]
    b, c = y.shape
    h = jax.lax.dot_general(y, w1_ref[...], (((1,), (1,)), ((), ())),
                            preferred_element_type=jnp.float32)
    h = jnp.maximum(h, 0.0)
    z = jax.lax.dot_general(h, w2_ref[...], (((1,), (1,)), ((), ())),
                            preferred_element_type=jnp.float32)
    g = jax.nn.sigmoid(z)

    gbits = jax.lax.bitcast_convert_type(g, jnp.int32)
    lo = jnp.zeros((b, 1), jnp.int32)
    hi = jnp.full((b, 1), 0x3F800000, jnp.int32)

    def body(_, lohi):
        lo, hi = lohi
        mid = lo + (hi - lo + 1) // 2
        cnt = jnp.sum((gbits >= mid).astype(jnp.int32), axis=1, keepdims=True)
        ok = cnt >= kkeep
        return jnp.where(ok, mid, lo), jnp.where(ok, hi, mid - 1)

    lo, hi = jax.lax.fori_loop(0, 31, body, (lo, hi))
    vbits = lo

    gt = (gbits > vbits).astype(jnp.float32)
    eq = (gbits == vbits).astype(jnp.float32)
    n_gt = jnp.sum(gt, axis=1, keepdims=True)
    need = jnp.float32(kkeep) - n_gt

    ii = jax.lax.broadcasted_iota(jnp.int32, (c, c), 0)
    jj = jax.lax.broadcasted_iota(jnp.int32, (c, c), 1)
    ltri = (ii < jj).astype(jnp.float32)
    prefix = jax.lax.dot_general(eq, ltri, (((1,), (0,)), ((), ())),
                                 preferred_element_type=jnp.float32)
    keep = gt + eq * (prefix < need).astype(jnp.float32)
    o_ref[...] = g * keep


def _scale_body(x_ref, g_ref, o_ref):
    o_ref[...] = x_ref[...] * g_ref[...]


def kernel(x, W1, W2):
    b, c, h, w = x.shape
    hw = h * w
    bc = b * c
    kkeep = c // 2
    xv = x.reshape(bc, h, w)

    sums = pl.pallas_call(
        _rowsum_body,
        grid=(bc // _ROWS_BLK,),
        in_specs=[pl.BlockSpec((_ROWS_BLK, h, w), lambda i: (i, 0, 0))],
        out_specs=pl.BlockSpec((_ROWS_BLK, 1, 1), lambda i: (i, 0, 0)),
        out_shape=jax.ShapeDtypeStruct((bc, 1, 1), jnp.float32),
    )(xv)
    means = sums.reshape(b, c) * (1.0 / hw)

    gate = pl.pallas_call(
        lambda *refs: _gate_body(kkeep, *refs),
        in_specs=[pl.BlockSpec((b, c), lambda: (0, 0)),
                  pl.BlockSpec(W1.shape, lambda: (0, 0)),
                  pl.BlockSpec(W2.shape, lambda: (0, 0))],
        out_specs=pl.BlockSpec((b, c), lambda: (0, 0)),
        out_shape=jax.ShapeDtypeStruct((b, c), jnp.float32),
    )(means, W1, W2)

    out = pl.pallas_call(
        _scale_body,
        grid=(bc // _ROWS_BLK,),
        in_specs=[pl.BlockSpec((_ROWS_BLK, h, w), lambda i: (i, 0, 0)),
                  pl.BlockSpec((_ROWS_BLK, 1, 1), lambda i: (i, 0, 0))],
        out_specs=pl.BlockSpec((_ROWS_BLK, h, w), lambda i: (i, 0, 0)),
        out_shape=jax.ShapeDtypeStruct((bc, h, w), jnp.float32),
    )(xv, gate.reshape(bc, 1, 1))

    return out.reshape(b, c, h, w)


# speedup vs baseline: 5.4224x; 2.9554x over previous
import jax
import jax.numpy as jnp
from jax.experimental import pallas as pl


_H_BLK = 16


def _colsum_body(x_ref, o_ref):
    s = jnp.sum(x_ref[...], axis=(1, 2), keepdims=True)

    @pl.when(pl.program_id(1) == 0)
    def _init():
        o_ref[...] = s

    @pl.when(pl.program_id(1) != 0)
    def _acc():
        o_ref[...] += s


def _gate_body(kkeep, m_ref, w1_ref, w2t_ref, o_ref):
    y = m_ref[...]
    b, c = y.shape
    h = jax.lax.dot_general(y, w1_ref[...], (((1,), (1,)), ((), ())),
                            preferred_element_type=jnp.float32)
    h = jnp.maximum(h, 0.0)
    z = jax.lax.dot_general(h, w2t_ref[...], (((1,), (0,)), ((), ())),
                            preferred_element_type=jnp.float32)
    g = jax.nn.sigmoid(z)

    gbits = jax.lax.bitcast_convert_type(g, jnp.int32)
    lo = jnp.zeros((b, 1), jnp.int32)
    hi = jnp.full((b, 1), 0x3F800000, jnp.int32)

    def body(_, lohi):
        lo, hi = lohi
        mid = lo + (hi - lo + 1) // 2
        cnt = jnp.sum((gbits >= mid).astype(jnp.int32), axis=1, keepdims=True)
        ok = cnt >= kkeep
        return jnp.where(ok, mid, lo), jnp.where(ok, hi, mid - 1)

    lo, hi = jax.lax.fori_loop(0, 31, body, (lo, hi))
    vbits = lo

    gt = (gbits > vbits).astype(jnp.float32)
    eq = (gbits == vbits).astype(jnp.float32)
    n_gt = jnp.sum(gt, axis=1, keepdims=True)
    need = jnp.float32(kkeep) - n_gt

    ii = jax.lax.broadcasted_iota(jnp.int32, (c, c), 0)
    jj = jax.lax.broadcasted_iota(jnp.int32, (c, c), 1)
    ltri = (ii < jj).astype(jnp.float32)
    prefix = jax.lax.dot_general(eq, ltri, (((1,), (0,)), ((), ())),
                                 preferred_element_type=jnp.float32)
    keep = gt + eq * (prefix < need).astype(jnp.float32)
    o_ref[...] = g * keep


def _scale_body(x_ref, g_ref, o_ref):
    o_ref[...] = x_ref[...] * g_ref[...]


def kernel(x, W1, W2):
    b, c, h, w = x.shape
    hw = h * w
    kkeep = c // 2
    hb = _H_BLK
    xt = jnp.transpose(x, (0, 2, 3, 1))

    sums = pl.pallas_call(
        _colsum_body,
        grid=(b, h // hb),
        in_specs=[pl.BlockSpec((1, hb, w, c), lambda i, j: (i, j, 0, 0))],
        out_specs=pl.BlockSpec((1, 1, 1, c), lambda i, j: (i, 0, 0, 0)),
        out_shape=jax.ShapeDtypeStruct((b, 1, 1, c), jnp.float32),
    )(xt)
    means = sums.reshape(b, c) * (1.0 / hw)

    gate = pl.pallas_call(
        lambda *refs: _gate_body(kkeep, *refs),
        in_specs=[pl.BlockSpec((b, c), lambda: (0, 0)),
                  pl.BlockSpec(W1.shape, lambda: (0, 0)),
                  pl.BlockSpec((W2.shape[1], W2.shape[0]), lambda: (0, 0))],
        out_specs=pl.BlockSpec((b, c), lambda: (0, 0)),
        out_shape=jax.ShapeDtypeStruct((b, c), jnp.float32),
    )(means, W1, W2.T)

    out_t = pl.pallas_call(
        _scale_body,
        grid=(b, h // hb),
        in_specs=[pl.BlockSpec((1, hb, w, c), lambda i, j: (i, j, 0, 0)),
                  pl.BlockSpec((1, 1, 1, c), lambda i, j: (i, 0, 0, 0))],
        out_specs=pl.BlockSpec((1, hb, w, c), lambda i, j: (i, j, 0, 0)),
        out_shape=jax.ShapeDtypeStruct((b, h, w, c), jnp.float32),
    )(xt, gate.reshape(b, 1, 1, c))

    return jnp.transpose(out_t, (0, 3, 1, 2))


# speedup vs baseline: 5.4747x; 1.0096x over previous
import jax
import jax.numpy as jnp
from jax.experimental import pallas as pl


_H_BLK = 28


def _colsum_body(x_ref, o_ref):
    s = jnp.sum(x_ref[...], axis=(1, 2), keepdims=True)

    @pl.when(pl.program_id(1) == 0)
    def _init():
        o_ref[...] = s

    @pl.when(pl.program_id(1) != 0)
    def _acc():
        o_ref[...] += s


def _gate_body(kkeep, m_ref, w1_ref, w2t_ref, o_ref):
    y = m_ref[...]
    b, c = y.shape
    h = jax.lax.dot_general(y, w1_ref[...], (((1,), (1,)), ((), ())),
                            preferred_element_type=jnp.float32)
    h = jnp.maximum(h, 0.0)
    z = jax.lax.dot_general(h, w2t_ref[...], (((1,), (0,)), ((), ())),
                            preferred_element_type=jnp.float32)
    g = jax.nn.sigmoid(z)

    gbits = jax.lax.bitcast_convert_type(g, jnp.int32)
    lo = jnp.zeros((b, 1), jnp.int32)
    hi = jnp.full((b, 1), 0x3F800000, jnp.int32)

    def body(_, lohi):
        lo, hi = lohi
        mid = lo + (hi - lo + 1) // 2
        cnt = jnp.sum((gbits >= mid).astype(jnp.int32), axis=1, keepdims=True)
        ok = cnt >= kkeep
        return jnp.where(ok, mid, lo), jnp.where(ok, hi, mid - 1)

    lo, hi = jax.lax.fori_loop(0, 31, body, (lo, hi))
    vbits = lo

    gt = (gbits > vbits).astype(jnp.float32)
    eq = (gbits == vbits).astype(jnp.float32)
    n_gt = jnp.sum(gt, axis=1, keepdims=True)
    need = jnp.float32(kkeep) - n_gt

    ii = jax.lax.broadcasted_iota(jnp.int32, (c, c), 0)
    jj = jax.lax.broadcasted_iota(jnp.int32, (c, c), 1)
    ltri = (ii < jj).astype(jnp.float32)
    prefix = jax.lax.dot_general(eq, ltri, (((1,), (0,)), ((), ())),
                                 preferred_element_type=jnp.float32)
    keep = gt + eq * (prefix < need).astype(jnp.float32)
    o_ref[...] = g * keep


def _scale_body(x_ref, g_ref, o_ref):
    o_ref[...] = x_ref[...] * g_ref[...]


def kernel(x, W1, W2):
    b, c, h, w = x.shape
    hw = h * w
    kkeep = c // 2
    hb = _H_BLK
    xt = jnp.transpose(x, (0, 2, 3, 1))

    sums = pl.pallas_call(
        _colsum_body,
        grid=(b, h // hb),
        in_specs=[pl.BlockSpec((1, hb, w, c), lambda i, j: (i, j, 0, 0))],
        out_specs=pl.BlockSpec((1, 1, 1, c), lambda i, j: (i, 0, 0, 0)),
        out_shape=jax.ShapeDtypeStruct((b, 1, 1, c), jnp.float32),
    )(xt)
    means = sums.reshape(b, c) * (1.0 / hw)

    gate = pl.pallas_call(
        lambda *refs: _gate_body(kkeep, *refs),
        in_specs=[pl.BlockSpec((b, c), lambda: (0, 0)),
                  pl.BlockSpec(W1.shape, lambda: (0, 0)),
                  pl.BlockSpec((W2.shape[1], W2.shape[0]), lambda: (0, 0))],
        out_specs=pl.BlockSpec((b, c), lambda: (0, 0)),
        out_shape=jax.ShapeDtypeStruct((b, c), jnp.float32),
    )(means, W1, W2.T)

    out_t = pl.pallas_call(
        _scale_body,
        grid=(b, h // hb),
        in_specs=[pl.BlockSpec((1, hb, w, c), lambda i, j: (i, j, 0, 0)),
                  pl.BlockSpec((1, 1, 1, c), lambda i, j: (i, 0, 0, 0))],
        out_specs=pl.BlockSpec((1, hb, w, c), lambda i, j: (i, j, 0, 0)),
        out_shape=jax.ShapeDtypeStruct((b, h, w, c), jnp.float32),
    )(xt, gate.reshape(b, 1, 1, c))

    return jnp.transpose(out_t, (0, 3, 1, 2))


# speedup vs baseline: 5.4959x; 1.0039x over previous
import jax
import jax.numpy as jnp
from jax.experimental import pallas as pl


_H_BLK = 32


def _colsum_body(x_ref, o_ref):
    s = jnp.sum(x_ref[...], axis=(1, 2), keepdims=True)

    @pl.when(pl.program_id(1) == 0)
    def _init():
        o_ref[...] = s

    @pl.when(pl.program_id(1) != 0)
    def _acc():
        o_ref[...] += s


def _gate_body(kkeep, m_ref, w1_ref, w2t_ref, o_ref):
    y = m_ref[...]
    b, c = y.shape
    h = jax.lax.dot_general(y, w1_ref[...], (((1,), (1,)), ((), ())),
                            preferred_element_type=jnp.float32)
    h = jnp.maximum(h, 0.0)
    z = jax.lax.dot_general(h, w2t_ref[...], (((1,), (0,)), ((), ())),
                            preferred_element_type=jnp.float32)
    g = jax.nn.sigmoid(z)

    gbits = jax.lax.bitcast_convert_type(g, jnp.int32)
    lo = jnp.zeros((b, 1), jnp.int32)
    hi = jnp.full((b, 1), 0x3F800000, jnp.int32)

    def body(_, lohi):
        lo, hi = lohi
        mid = lo + (hi - lo + 1) // 2
        cnt = jnp.sum((gbits >= mid).astype(jnp.int32), axis=1, keepdims=True)
        ok = cnt >= kkeep
        return jnp.where(ok, mid, lo), jnp.where(ok, hi, mid - 1)

    lo, hi = jax.lax.fori_loop(0, 31, body, (lo, hi))
    vbits = lo

    gt = (gbits > vbits).astype(jnp.float32)
    eq = (gbits == vbits).astype(jnp.float32)
    n_gt = jnp.sum(gt, axis=1, keepdims=True)
    need = jnp.float32(kkeep) - n_gt

    ii = jax.lax.broadcasted_iota(jnp.int32, (c, c), 0)
    jj = jax.lax.broadcasted_iota(jnp.int32, (c, c), 1)
    ltri = (ii < jj).astype(jnp.float32)
    prefix = jax.lax.dot_general(eq, ltri, (((1,), (0,)), ((), ())),
                                 preferred_element_type=jnp.float32)
    keep = gt + eq * (prefix < need).astype(jnp.float32)
    o_ref[...] = g * keep


def _scale_body(x_ref, g_ref, o_ref):
    o_ref[...] = x_ref[...] * g_ref[...]


def kernel(x, W1, W2):
    b, c, h, w = x.shape
    hw = h * w
    kkeep = c // 2
    hb = _H_BLK
    xt = jnp.transpose(x, (0, 2, 3, 1))

    sums = pl.pallas_call(
        _colsum_body,
        grid=(b, h // hb),
        in_specs=[pl.BlockSpec((1, hb, w, c), lambda i, j: (i, j, 0, 0))],
        out_specs=pl.BlockSpec((1, 1, 1, c), lambda i, j: (i, 0, 0, 0)),
        out_shape=jax.ShapeDtypeStruct((b, 1, 1, c), jnp.float32),
    )(xt)
    means = sums.reshape(b, c) * (1.0 / hw)

    gate = pl.pallas_call(
        lambda *refs: _gate_body(kkeep, *refs),
        in_specs=[pl.BlockSpec((b, c), lambda: (0, 0)),
                  pl.BlockSpec(W1.shape, lambda: (0, 0)),
                  pl.BlockSpec((W2.shape[1], W2.shape[0]), lambda: (0, 0))],
        out_specs=pl.BlockSpec((b, c), lambda: (0, 0)),
        out_shape=jax.ShapeDtypeStruct((b, c), jnp.float32),
    )(means, W1, W2.T)

    out_t = pl.pallas_call(
        _scale_body,
        grid=(b, h // hb),
        in_specs=[pl.BlockSpec((1, hb, w, c), lambda i, j: (i, j, 0, 0)),
                  pl.BlockSpec((1, 1, 1, c), lambda i, j: (i, 0, 0, 0))],
        out_specs=pl.BlockSpec((1, hb, w, c), lambda i, j: (i, j, 0, 0)),
        out_shape=jax.ShapeDtypeStruct((b, h, w, c), jnp.float32),
    )(xt, gate.reshape(b, 1, 1, c))

    return jnp.transpose(out_t, (0, 3, 1, 2))
